# baseline (speedup 1.0000x reference)
import functools

import jax
import jax.numpy as jnp
from jax import lax
from jax.experimental import pallas as pl
from jax.experimental.pallas import tpu as pltpu
from jax.experimental.pallas import tpu_sc as plsc

BATCH = 16384
HIST = 50
DIM = 32
NUM_CORES = 2
NUM_SUBCORES = 16
NW = NUM_CORES * NUM_SUBCORES
BTILE = 128
NBT = BATCH // BTILE
UNITS = HIST * NBT
PER_W = UNITS // NW
NB = 4
NGROUP = PER_W // NB

_mesh = plsc.VectorSubcoreMesh(core_axis_name="c", subcore_axis_name="s")


@functools.partial(
    pl.kernel,
    mesh=_mesh,
    out_type=jax.ShapeDtypeStruct((HIST, DIM // 8, NBT, 8, BTILE), jnp.float32),
    scratch_types=[
        pltpu.VMEM((NB, BTILE), jnp.int32),
        pltpu.VMEM((NB, BTILE, DIM), jnp.float32),
        pltpu.VMEM((NB, DIM, BTILE), jnp.float32),
        pltpu.SemaphoreType.DMA((NB,)),
        pltpu.SemaphoreType.DMA((NB,)),
        pltpu.SemaphoreType.DMA((NB,)),
    ],
    compiler_params=pltpu.CompilerParams(
        use_tc_tiling_on_sc=False, needs_layout_passes=False,
        disable_bounds_checks=True),
)
def _gather_kernel(idx_hbm, table_hbm, out_hbm, idx_v, bufa, buft, isem, gsem, osem):
    wid = lax.axis_index("s") * NUM_CORES + lax.axis_index("c")
    u0 = wid * PER_W

    def hc(u):
        return u // NBT, lax.rem(u, NBT)

    def idx_desc(u, b):
        h, c = hc(u)
        return pltpu.make_async_copy(
            idx_hbm.at[h, pl.ds(c * BTILE, BTILE)], idx_v.at[b], isem.at[b])

    def gather_desc(b):
        return pltpu.make_async_copy(
            table_hbm.at[idx_v.at[b]], bufa.at[b], gsem.at[b])

    def out_desc(u, s, b):
        h, c = hc(u)
        return pltpu.make_async_copy(
            buft.at[b, pl.ds(s * 8, 8)], out_hbm.at[h, s, c], osem.at[b])

    row_vecs = [jnp.arange(eb * 16, eb * 16 + 16, dtype=jnp.int32)
                for eb in range(BTILE // 16)]

    def transpose(b):
        src = bufa.at[b]
        dst = buft.at[b]

        def body(f0, carry):
            cols = (jnp.full((16,), f0, jnp.int32) + lax.iota(jnp.int32, 16)) & 31
            for eb in range(BTILE // 16):
                rows = row_vecs[eb]
                vals = plsc.load_gather(src, [rows, cols])
                plsc.store_scatter(dst, [cols, rows], vals)
            return carry

        lax.fori_loop(0, DIM, body, 0, unroll=4)

    def step(u, b, drain, gather_ahead, idx_ahead):
        bg = (b + 2) % NB
        if gather_ahead:
            idx_desc(u + 2, bg).wait()
            gather_desc(bg).start()
        if drain:
            for s in range(DIM // 8):
                out_desc(u - NB, s, b).wait()
        gather_desc(b).wait()
        if idx_ahead:
            idx_desc(u + NB, b).start()
        transpose(b)
        for s in range(DIM // 8):
            out_desc(u, s, b).start()

    for b in range(NB):
        idx_desc(u0 + b, b).start()
    for b in range(2):
        idx_desc(u0 + b, b).wait()
        gather_desc(b).start()

    for b in range(NB):
        step(u0 + b, b, drain=False, gather_ahead=True, idx_ahead=True)

    def group(g, carry):
        ug = u0 + g * NB
        for b in range(NB):
            step(ug + b, b, drain=True, gather_ahead=True, idx_ahead=True)
        return carry

    lax.fori_loop(1, NGROUP - 1, group, 0, unroll=False)

    ul = u0 + (NGROUP - 1) * NB
    for b in range(NB):
        step(ul + b, b, drain=True, gather_ahead=(b < 2), idx_ahead=False)

    for b in range(NB):
        for s in range(DIM // 8):
            out_desc(ul + b, s, b).wait()


def kernel(substructure_indices, embedding_table):
    idx_t = substructure_indices.T
    out5 = _gather_kernel(idx_t, embedding_table)
    return out5.transpose(2, 4, 0, 1, 3).reshape(BATCH, HIST, DIM)


# speedup vs baseline: 1.0021x; 1.0021x over previous
import functools

import jax
import jax.numpy as jnp
from jax import lax
from jax.experimental import pallas as pl
from jax.experimental.pallas import tpu as pltpu
from jax.experimental.pallas import tpu_sc as plsc

BATCH = 16384
HIST = 50
DIM = 32
NUM_CORES = 2
NUM_SUBCORES = 16
NW = NUM_CORES * NUM_SUBCORES
BTILE = 128
NBT = BATCH // BTILE
UNITS = HIST * NBT
PER_W = UNITS // NW
NB = 4
NGROUP = PER_W // NB

_mesh = plsc.VectorSubcoreMesh(core_axis_name="c", subcore_axis_name="s")


@functools.partial(
    pl.kernel,
    mesh=_mesh,
    out_type=jax.ShapeDtypeStruct((HIST, DIM // 8, NBT, 8, BTILE), jnp.float32),
    scratch_types=[
        pltpu.VMEM((NB, BTILE), jnp.int32),
        pltpu.VMEM((NB, BTILE, DIM), jnp.float32),
        pltpu.VMEM((NB, DIM, BTILE), jnp.float32),
        pltpu.SemaphoreType.DMA((NB,)),
        pltpu.SemaphoreType.DMA((NB,)),
        pltpu.SemaphoreType.DMA((NB,)),
    ],
    compiler_params=pltpu.CompilerParams(
        use_tc_tiling_on_sc=False, needs_layout_passes=False,
        disable_bounds_checks=True),
)
def _gather_kernel(idx_hbm, table_hbm, out_hbm, idx_v, bufa, buft, isem, gsem, osem):
    wid = lax.axis_index("s") * NUM_CORES + lax.axis_index("c")
    u0 = wid * PER_W

    def hc(u):
        return u // NBT, lax.rem(u, NBT)

    def idx_desc(u, b):
        h, c = hc(u)
        return pltpu.make_async_copy(
            idx_hbm.at[h, pl.ds(c * BTILE, BTILE)], idx_v.at[b], isem.at[b])

    def gather_desc(b):
        return pltpu.make_async_copy(
            table_hbm.at[idx_v.at[b]], bufa.at[b], gsem.at[b])

    def out_desc(u, s, b):
        h, c = hc(u)
        return pltpu.make_async_copy(
            buft.at[b, pl.ds(s * 8, 8)], out_hbm.at[h, s, c], osem.at[b])

    col_vecs = [(jnp.arange(f0, f0 + 16, dtype=jnp.int32)) & 31
                for f0 in range(DIM)]

    def transpose(b):
        src = bufa.at[b]
        dst = buft.at[b]

        def body(eb, carry):
            rows = lax.iota(jnp.int32, 16) + jnp.full((16,), eb * 16, jnp.int32)
            for f0 in range(DIM):
                cols = col_vecs[f0]
                vals = plsc.load_gather(src, [rows, cols])
                plsc.store_scatter(dst, [cols, rows], vals)
            return carry

        lax.fori_loop(0, BTILE // 16, body, 0, unroll=False)

    def step(u, b, drain, gather_ahead, idx_ahead):
        bg = (b + 2) % NB
        if gather_ahead:
            idx_desc(u + 2, bg).wait()
            gather_desc(bg).start()
        if drain:
            for s in range(DIM // 8):
                out_desc(u - NB, s, b).wait()
        gather_desc(b).wait()
        if idx_ahead:
            idx_desc(u + NB, b).start()
        transpose(b)
        for s in range(DIM // 8):
            out_desc(u, s, b).start()

    for b in range(NB):
        idx_desc(u0 + b, b).start()
    for b in range(2):
        idx_desc(u0 + b, b).wait()
        gather_desc(b).start()

    for b in range(NB):
        step(u0 + b, b, drain=False, gather_ahead=True, idx_ahead=True)

    def group(g, carry):
        ug = u0 + g * NB
        for b in range(NB):
            step(ug + b, b, drain=True, gather_ahead=True, idx_ahead=True)
        return carry

    lax.fori_loop(1, NGROUP - 1, group, 0, unroll=False)

    ul = u0 + (NGROUP - 1) * NB
    for b in range(NB):
        step(ul + b, b, drain=True, gather_ahead=(b < 2), idx_ahead=False)

    for b in range(NB):
        for s in range(DIM // 8):
            out_desc(ul + b, s, b).wait()


def kernel(substructure_indices, embedding_table):
    idx_t = substructure_indices.T
    out5 = _gather_kernel(idx_t, embedding_table)
    return out5.transpose(2, 4, 0, 1, 3).reshape(BATCH, HIST, DIM)


# speedup vs baseline: 1.0123x; 1.0102x over previous
import functools

import jax
import jax.numpy as jnp
from jax import lax
from jax.experimental import pallas as pl
from jax.experimental.pallas import tpu as pltpu
from jax.experimental.pallas import tpu_sc as plsc

BATCH = 16384
HIST = 50
DIM = 32
NUM_CORES = 2
NUM_SUBCORES = 16
NW = NUM_CORES * NUM_SUBCORES
BTILE = 128
NBT = BATCH // BTILE
UNITS = HIST * NBT
PER_W = UNITS // NW
NB = 4
NGROUP = PER_W // NB

_mesh = plsc.VectorSubcoreMesh(core_axis_name="c", subcore_axis_name="s")


@functools.partial(
    pl.kernel,
    mesh=_mesh,
    out_type=jax.ShapeDtypeStruct((HIST, DIM // 8, NBT, 8, BTILE), jnp.float32),
    scratch_types=[
        pltpu.VMEM((NB, BTILE), jnp.int32),
        pltpu.VMEM((NB, BTILE, DIM), jnp.float32),
        pltpu.VMEM((NB, DIM, BTILE), jnp.float32),
        pltpu.SemaphoreType.DMA((NB,)),
        pltpu.SemaphoreType.DMA((NB,)),
        pltpu.SemaphoreType.DMA((NB,)),
    ],
    compiler_params=pltpu.CompilerParams(
        use_tc_tiling_on_sc=False, needs_layout_passes=False,
        disable_bounds_checks=True),
)
def _gather_kernel(idx_hbm, table_hbm, out_hbm, idx_v, bufa, buft, isem, gsem, osem):
    wid = lax.axis_index("s") * NUM_CORES + lax.axis_index("c")
    u0 = wid * PER_W

    def hc(u):
        return u // NBT, lax.rem(u, NBT)

    def idx_desc(u, b):
        h, c = hc(u)
        return pltpu.make_async_copy(
            idx_hbm.at[h, pl.ds(c * BTILE, BTILE)], idx_v.at[b], isem.at[b])

    def gather_desc(b):
        return pltpu.make_async_copy(
            table_hbm.at[idx_v.at[b]], bufa.at[b], gsem.at[b])

    def out_desc(u, s, b):
        h, c = hc(u)
        return pltpu.make_async_copy(
            buft.at[b, pl.ds(s * 8, 8)], out_hbm.at[h, s, c], osem.at[b])


    def transpose(b):
        src = bufa.at[b]
        dst = buft.at[b]

        def body(f0, carry):
            cols = (jnp.full((16,), f0, jnp.int32) + lax.iota(jnp.int32, 16)) & 31
            for eb in range(BTILE // 16):
                rows = jnp.arange(eb * 16, eb * 16 + 16, dtype=jnp.int32)
                vals = plsc.load_gather(src, [rows, cols])
                plsc.store_scatter(dst, [cols, rows], vals)
            return carry

        lax.fori_loop(0, DIM, body, 0, unroll=4)

    def step(u, b, drain, gather_ahead, idx_ahead):
        bg = (b + 2) % NB
        if gather_ahead:
            idx_desc(u + 2, bg).wait()
            gather_desc(bg).start()
        if drain:
            for s in range(DIM // 8):
                out_desc(u - NB, s, b).wait()
        gather_desc(b).wait()
        if idx_ahead:
            idx_desc(u + NB, b).start()
        transpose(b)
        for s in range(DIM // 8):
            out_desc(u, s, b).start()

    for b in range(NB):
        idx_desc(u0 + b, b).start()
    for b in range(2):
        idx_desc(u0 + b, b).wait()
        gather_desc(b).start()

    for b in range(NB):
        step(u0 + b, b, drain=False, gather_ahead=True, idx_ahead=True)

    def group(g, carry):
        ug = u0 + g * NB
        for b in range(NB):
            step(ug + b, b, drain=True, gather_ahead=True, idx_ahead=True)
        return carry

    lax.fori_loop(1, NGROUP - 1, group, 0, unroll=False)

    ul = u0 + (NGROUP - 1) * NB
    for b in range(NB):
        step(ul + b, b, drain=True, gather_ahead=(b < 2), idx_ahead=False)

    for b in range(NB):
        for s in range(DIM // 8):
            out_desc(ul + b, s, b).wait()


def kernel(substructure_indices, embedding_table):
    idx_t = substructure_indices.T
    out5 = _gather_kernel(idx_t, embedding_table)
    return out5.transpose(2, 4, 0, 1, 3).reshape(BATCH, HIST, DIM)
